# baseline (speedup 1.0000x reference)
import functools

import jax
import jax.numpy as jnp
import numpy as np
from jax import lax
from jax.experimental import pallas as pl
from jax.experimental.pallas import tpu as pltpu
from jax.experimental.pallas import tpu_sc as plsc

_B = 16384
_F = 26
_NW = 32
_RPW = _B // _NW
_CHUNK = _F * _RPW
_HALF = _CHUNK // 2


def _make_kernel():
    mesh = plsc.VectorSubcoreMesh(core_axis_name="c", subcore_axis_name="s")

    @functools.partial(
        pl.kernel,
        mesh=mesh,
        out_type=jax.ShapeDtypeStruct((_B,), jnp.float32),
        compiler_params=pltpu.CompilerParams(needs_layout_passes=False),
        scratch_types=[
            pltpu.VMEM((_CHUNK,), jnp.int32),
            pltpu.VMEM((_CHUNK,), jnp.float32),
            pltpu.VMEM((_RPW,), jnp.float32),
            pltpu.SemaphoreType.DMA,
            pltpu.SemaphoreType.DMA,
            pltpu.SemaphoreType.DMA,
        ],
    )
    def k(idx_hbm, w_hbm, out_hbm, idx_v, vals_v, out_v, si, s0, s1):
        wid = lax.axis_index("s") * 2 + lax.axis_index("c")
        w_flat = w_hbm.at[0]
        base = wid * _CHUNK
        pltpu.async_copy(
            idx_hbm.at[pl.ds(base, _HALF)], idx_v.at[pl.ds(0, _HALF)], si
        ).wait()
        g0 = pltpu.async_copy(
            w_flat.at[idx_v.at[pl.ds(0, _HALF)]], vals_v.at[pl.ds(0, _HALF)], s0
        )
        pltpu.async_copy(
            idx_hbm.at[pl.ds(base + _HALF, _HALF)],
            idx_v.at[pl.ds(_HALF, _HALF)],
            si,
        ).wait()
        g1 = pltpu.async_copy(
            w_flat.at[idx_v.at[pl.ds(_HALF, _HALF)]],
            vals_v.at[pl.ds(_HALF, _HALF)],
            s1,
        )
        zero = jnp.zeros((16,), jnp.float32)
        for g in range(_RPW // 16):
            out_v[pl.ds(g * 16, 16)] = zero
        lanes = lax.iota(jnp.int32, 16)
        g0.wait()
        for g in range(_HALF // 16):
            rows = (lanes + g * 16) // _F
            plsc.addupdate_scatter(out_v, [rows], vals_v[pl.ds(g * 16, 16)])
        g1.wait()
        for g in range(_HALF // 16, _CHUNK // 16):
            rows = (lanes + g * 16) // _F
            plsc.addupdate_scatter(out_v, [rows], vals_v[pl.ds(g * 16, 16)])
        pltpu.sync_copy(out_v, out_hbm.at[pl.ds(wid * _RPW, _RPW)])

    return k


_sc_kernel = _make_kernel()


def kernel(inputs, w):
    idx = inputs.astype(jnp.int32).reshape(-1)
    out = _sc_kernel(idx, w.T)
    return out.reshape(_B, 1)


# speedup vs baseline: 1.6307x; 1.6307x over previous
import functools

import jax
import jax.numpy as jnp
from jax import lax
from jax.experimental import pallas as pl
from jax.experimental.pallas import tpu as pltpu
from jax.experimental.pallas import tpu_sc as plsc

_B = 16384
_F = 26
_NW = 32
_RPW = _B // _NW
_CHUNK = _F * _RPW
_FH = _F // 2
_IPH = _FH * _RPW


def _make_kernel():
    mesh = plsc.VectorSubcoreMesh(core_axis_name="c", subcore_axis_name="s")

    @functools.partial(
        pl.kernel,
        mesh=mesh,
        out_type=jax.ShapeDtypeStruct((_B,), jnp.float32),
        compiler_params=pltpu.CompilerParams(needs_layout_passes=False),
        scratch_types=[
            pltpu.VMEM((_CHUNK,), jnp.int32),
            pltpu.VMEM((_CHUNK,), jnp.float32),
            pltpu.VMEM((_RPW,), jnp.float32),
            pltpu.SemaphoreType.DMA,
            pltpu.SemaphoreType.DMA,
            pltpu.SemaphoreType.DMA,
        ],
    )
    def k(idx_hbm, w_hbm, out_hbm, idx_v, vals_v, out_v, si, s0, s1):
        wid = lax.axis_index("s") * 2 + lax.axis_index("c")
        w_flat = w_hbm.at[0]
        tile_idx = idx_hbm.at[wid]
        pltpu.async_copy(
            tile_idx.at[pl.ds(0, _IPH)], idx_v.at[pl.ds(0, _IPH)], si
        ).wait()
        g0 = pltpu.async_copy(
            w_flat.at[idx_v.at[pl.ds(0, _IPH)]], vals_v.at[pl.ds(0, _IPH)], s0
        )
        pltpu.async_copy(
            tile_idx.at[pl.ds(_IPH, _IPH)], idx_v.at[pl.ds(_IPH, _IPH)], si
        ).wait()
        g1 = pltpu.async_copy(
            w_flat.at[idx_v.at[pl.ds(_IPH, _IPH)]],
            vals_v.at[pl.ds(_IPH, _IPH)],
            s1,
        )
        g0.wait()
        for g in range(_RPW // 16):
            acc = vals_v[pl.ds(g * 16, 16)]
            for f in range(1, _FH):
                acc = acc + vals_v[pl.ds(f * _RPW + g * 16, 16)]
            out_v[pl.ds(g * 16, 16)] = acc
        g1.wait()
        for g in range(_RPW // 16):
            acc = out_v[pl.ds(g * 16, 16)]
            for f in range(_FH):
                acc = acc + vals_v[pl.ds(_IPH + f * _RPW + g * 16, 16)]
            out_v[pl.ds(g * 16, 16)] = acc
        pltpu.sync_copy(out_v, out_hbm.at[pl.ds(wid * _RPW, _RPW)])

    return k


_sc_kernel = _make_kernel()


def kernel(inputs, w):
    idx = inputs.astype(jnp.int32).reshape(_NW, _RPW, _F).transpose(0, 2, 1)
    out = _sc_kernel(idx.reshape(_NW, _CHUNK), w.T)
    return out.reshape(_B, 1)


# speedup vs baseline: 1.6735x; 1.0263x over previous
import functools

import jax
import jax.numpy as jnp
from jax import lax
from jax.experimental import pallas as pl
from jax.experimental.pallas import tpu as pltpu
from jax.experimental.pallas import tpu_sc as plsc

_B = 16384
_F = 26
_NW = 32
_RPW = _B // _NW
_CHUNK = _F * _RPW
_FH = _F // 2
_IPH = _FH * _RPW


def _make_kernel():
    mesh = plsc.VectorSubcoreMesh(core_axis_name="c", subcore_axis_name="s")

    @functools.partial(
        pl.kernel,
        mesh=mesh,
        out_type=jax.ShapeDtypeStruct((_B,), jnp.float32),
        compiler_params=pltpu.CompilerParams(needs_layout_passes=False),
        scratch_types=[
            pltpu.VMEM((_CHUNK,), jnp.int32),
            pltpu.VMEM((_CHUNK,), jnp.float32),
            pltpu.VMEM((_RPW,), jnp.float32),
            pltpu.SemaphoreType.DMA,
            pltpu.SemaphoreType.DMA,
            pltpu.SemaphoreType.DMA,
        ],
    )
    def k(idx_hbm, w_hbm, out_hbm, idx_v, vals_v, out_v, si, s0, s1):
        wid = lax.axis_index("s") * 2 + lax.axis_index("c")
        w_flat = w_hbm.at[0]
        tile_idx = idx_hbm.at[wid]
        pltpu.async_copy(
            tile_idx.at[pl.ds(0, _IPH)], idx_v.at[pl.ds(0, _IPH)], si
        ).wait()
        g0 = pltpu.async_copy(
            w_flat.at[idx_v.at[pl.ds(0, _IPH)]], vals_v.at[pl.ds(0, _IPH)], s0
        )
        pltpu.async_copy(
            tile_idx.at[pl.ds(_IPH, _IPH)], idx_v.at[pl.ds(_IPH, _IPH)], si
        ).wait()
        g1 = pltpu.async_copy(
            w_flat.at[idx_v.at[pl.ds(_IPH, _IPH)]],
            vals_v.at[pl.ds(_IPH, _IPH)],
            s1,
        )
        g0.wait()

        def red0(g, _):
            o = g * 16
            acc = vals_v[pl.ds(o, 16)]
            for f in range(1, _FH):
                acc = acc + vals_v[pl.ds(f * _RPW + o, 16)]
            out_v[pl.ds(o, 16)] = acc
            return 0

        lax.fori_loop(0, _RPW // 16, red0, 0, unroll=2)
        g1.wait()

        def red1(g, _):
            o = g * 16
            acc = out_v[pl.ds(o, 16)]
            for f in range(_FH):
                acc = acc + vals_v[pl.ds(_IPH + f * _RPW + o, 16)]
            out_v[pl.ds(o, 16)] = acc
            return 0

        lax.fori_loop(0, _RPW // 16, red1, 0, unroll=2)
        pltpu.sync_copy(out_v, out_hbm.at[pl.ds(wid * _RPW, _RPW)])

    return k


_sc_kernel = _make_kernel()


def kernel(inputs, w):
    idx = inputs.astype(jnp.int32).reshape(_NW, _RPW, _F).transpose(0, 2, 1)
    out = _sc_kernel(idx.reshape(_NW, _CHUNK), w.T)
    return out.reshape(_B, 1)


# speedup vs baseline: 1.6926x; 1.0114x over previous
import functools

import jax
import jax.numpy as jnp
from jax import lax
from jax.experimental import pallas as pl
from jax.experimental.pallas import tpu as pltpu
from jax.experimental.pallas import tpu_sc as plsc

_B = 16384
_F = 26
_NW = 32
_RPW = _B // _NW
_CHUNK = _F * _RPW
_FH = _F // 2
_IPH = _FH * _RPW


def _make_kernel():
    mesh = plsc.VectorSubcoreMesh(core_axis_name="c", subcore_axis_name="s")

    @functools.partial(
        pl.kernel,
        mesh=mesh,
        out_type=jax.ShapeDtypeStruct((_B,), jnp.float32),
        compiler_params=pltpu.CompilerParams(needs_layout_passes=False),
        scratch_types=[
            pltpu.VMEM((_CHUNK,), jnp.int32),
            pltpu.VMEM((_CHUNK,), jnp.float32),
            pltpu.VMEM((_RPW,), jnp.float32),
            pltpu.SemaphoreType.DMA,
            pltpu.SemaphoreType.DMA,
            pltpu.SemaphoreType.DMA,
        ],
    )
    def k(idx_hbm, w_hbm, out_hbm, idx_v, vals_v, out_v, si, s0, s1):
        wid = lax.axis_index("s") * 2 + lax.axis_index("c")
        w_flat = w_hbm.at[0]
        tile_idx = idx_hbm.at[wid]
        pltpu.async_copy(
            tile_idx.at[pl.ds(0, _IPH)], idx_v.at[pl.ds(0, _IPH)], si
        ).wait()
        g0 = pltpu.async_copy(
            w_flat.at[idx_v.at[pl.ds(0, _IPH)]], vals_v.at[pl.ds(0, _IPH)], s0
        )
        pltpu.async_copy(
            tile_idx.at[pl.ds(_IPH, _IPH)], idx_v.at[pl.ds(_IPH, _IPH)], si
        ).wait()
        g1 = pltpu.async_copy(
            w_flat.at[idx_v.at[pl.ds(_IPH, _IPH)]],
            vals_v.at[pl.ds(_IPH, _IPH)],
            s1,
        )
        g0.wait()

        @plsc.parallel_loop(0, _RPW, 16, unroll=2)
        def red0(o):
            acc = vals_v[pl.ds(o, 16)]
            for f in range(1, _FH):
                acc = acc + vals_v[pl.ds(f * _RPW + o, 16)]
            out_v[pl.ds(o, 16)] = acc

        g1.wait()

        @plsc.parallel_loop(0, _RPW, 16, unroll=2)
        def red1(o):
            acc = out_v[pl.ds(o, 16)]
            for f in range(_FH):
                acc = acc + vals_v[pl.ds(_IPH + f * _RPW + o, 16)]
            out_v[pl.ds(o, 16)] = acc
        pltpu.sync_copy(out_v, out_hbm.at[pl.ds(wid * _RPW, _RPW)])

    return k


_sc_kernel = _make_kernel()


def kernel(inputs, w):
    idx = inputs.astype(jnp.int32).reshape(_NW, _RPW, _F).transpose(0, 2, 1)
    out = _sc_kernel(idx.reshape(_NW, _CHUNK), w.T)
    return out.reshape(_B, 1)
